# baseline (speedup 1.0000x reference)
import functools

import jax
import jax.numpy as jnp
from jax import lax
from jax.experimental import pallas as pl
from jax.experimental.pallas import tpu as pltpu
from jax.experimental.pallas import tpu_sc as plsc

_NC = 2
_NS = 16
_NW = _NC * _NS

_IDX_CHUNK = 128

_TL = 32768


def _repack_body(tT_ref, out_ref):
  x = tT_ref[...]
  out_ref[...] = jnp.concatenate(
      [jnp.transpose(x[:, :_TL // 2]), jnp.transpose(x[:, _TL // 2:])], axis=1
  )


def _repack(tableT, V, D):
  grid = (V + _TL - 1) // _TL
  return pl.pallas_call(
      _repack_body,
      grid=(grid,),
      in_specs=[pl.BlockSpec((D, _TL), lambda i: (0, i))],
      out_specs=pl.BlockSpec((_TL // 2, 2 * D), lambda i: (i, 0)),
      out_shape=jax.ShapeDtypeStruct((grid * (_TL // 2), 2 * D), jnp.float32),
  )(tableT)


def _make_sc_gather(P, W, B):
  b_per_w = B // _NW
  n_chunks = b_per_w // _IDX_CHUNK
  mesh = plsc.VectorSubcoreMesh(core_axis_name="c", subcore_axis_name="s")

  @functools.partial(
      pl.kernel,
      mesh=mesh,
      out_type=jax.ShapeDtypeStruct((B, W), jnp.float32),
      scratch_types=[
          pltpu.VMEM((n_chunks, _IDX_CHUNK), jnp.int32),
          pltpu.VMEM((b_per_w, W), jnp.float32),
          pltpu.SemaphoreType.DMA,
      ],
  )
  def gather_k(packed_hbm, idx_hbm, out_hbm, idx_v, panel, sem):
    wid = lax.axis_index("s") * _NC + lax.axis_index("c")
    base = wid * b_per_w
    pltpu.sync_copy(idx_hbm.at[wid], idx_v)
    copies = []
    for j in range(n_chunks):
      copies.append(
          pltpu.async_copy(
              packed_hbm.at[idx_v.at[j]],
              panel.at[pl.ds(j * _IDX_CHUNK, _IDX_CHUNK)],
              sem,
          )
      )
    for c in copies:
      c.wait()
    pltpu.sync_copy(panel, out_hbm.at[pl.ds(base, b_per_w)])

  return gather_k


def _gru_body(emb2_ref, par_ref, h_ref, wi_ref, wh_ref, bi_ref, bh_ref,
              out_ref):
  emb2 = emb2_ref[...]
  par = par_ref[...]
  emb = jnp.where(par == 0, emb2[:, :64], emb2[:, 64:])
  h = h_ref[...]
  f32 = jnp.float32
  gi_r = jnp.dot(emb, wi_ref[0], preferred_element_type=f32) + bi_ref[0]
  gi_z = jnp.dot(emb, wi_ref[1], preferred_element_type=f32) + bi_ref[1]
  gi_n = jnp.dot(emb, wi_ref[2], preferred_element_type=f32) + bi_ref[2]
  gh_r = jnp.dot(h, wh_ref[0], preferred_element_type=f32) + bh_ref[0]
  gh_z = jnp.dot(h, wh_ref[1], preferred_element_type=f32) + bh_ref[1]
  gh_n = jnp.dot(h, wh_ref[2], preferred_element_type=f32) + bh_ref[2]
  r = jax.nn.sigmoid(gi_r + gh_r)
  z = jax.nn.sigmoid(gi_z + gh_z)
  n = jnp.tanh(gi_n + r * gh_n)
  out_ref[...] = (1.0 - z) * n + z * h


def _gru_call(emb2, par, h, wi, wh, bi, bh, blk):
  B, H = h.shape
  grid = B // blk
  return pl.pallas_call(
      _gru_body,
      grid=(grid,),
      in_specs=[
          pl.BlockSpec((blk, emb2.shape[1]), lambda i: (i, 0)),
          pl.BlockSpec((blk, 1), lambda i: (i, 0)),
          pl.BlockSpec((blk, H), lambda i: (i, 0)),
          pl.BlockSpec(wi.shape, lambda i: (0, 0, 0)),
          pl.BlockSpec(wh.shape, lambda i: (0, 0, 0)),
          pl.BlockSpec(bi.shape, lambda i: (0, 0, 0)),
          pl.BlockSpec(bh.shape, lambda i: (0, 0, 0)),
      ],
      out_specs=pl.BlockSpec((blk, H), lambda i: (i, 0)),
      out_shape=jax.ShapeDtypeStruct((B, H), jnp.float32),
  )(emb2, par, h, wi, wh, bi, bh)


@jax.jit
def kernel(x, h, table, W_ih, W_hh, b_ih, b_hh):
  V, D = table.shape
  B, H = h.shape

  packed = _repack(table.T, V, D)

  xi = x.astype(jnp.int32)
  half = _TL // 2
  b = xi // _TL
  q = xi % _TL
  idx2 = (b * half + q % half).reshape(
      _NW, (B // _NW) // _IDX_CHUNK, _IDX_CHUNK
  )
  par = (q // half).reshape(B, 1)

  emb2 = _make_sc_gather(packed.shape[0], 2 * D, B)(packed, idx2)

  wi = jnp.transpose(W_ih.reshape(3, H, D), (0, 2, 1))
  wh = jnp.transpose(W_hh.reshape(3, H, H), (0, 2, 1))
  bi = b_ih.reshape(3, 1, H)
  bh = b_hh.reshape(3, 1, H)

  return _gru_call(emb2, par, h, wi, wh, bi, bh, blk=2048)


# speedup vs baseline: 1.0051x; 1.0051x over previous
import functools

import jax
import jax.numpy as jnp
from jax import lax
from jax.experimental import pallas as pl
from jax.experimental.pallas import tpu as pltpu
from jax.experimental.pallas import tpu_sc as plsc

_NC = 2
_NS = 16
_NW = _NC * _NS

_IDX_CHUNK = 128

_TL = 32768


def _repack_body(tT_ref, out_ref):
  x = tT_ref[...]
  out_ref[...] = jnp.concatenate(
      [jnp.transpose(x[:, :_TL // 2]), jnp.transpose(x[:, _TL // 2:])], axis=1
  )


def _repack(tableT, V, D):
  grid = (V + _TL - 1) // _TL
  return pl.pallas_call(
      _repack_body,
      grid=(grid,),
      in_specs=[pl.BlockSpec((D, _TL), lambda i: (0, i))],
      out_specs=pl.BlockSpec((_TL // 2, 2 * D), lambda i: (i, 0)),
      out_shape=jax.ShapeDtypeStruct((grid * (_TL // 2), 2 * D), jnp.float32),
  )(tableT)


def _make_sc_gather(P, W, B):
  b_per_w = B // _NW
  n_chunks = b_per_w // _IDX_CHUNK
  mesh = plsc.VectorSubcoreMesh(core_axis_name="c", subcore_axis_name="s")

  @functools.partial(
      pl.kernel,
      mesh=mesh,
      out_type=jax.ShapeDtypeStruct((B, W), jnp.float32),
      scratch_types=[
          pltpu.VMEM((n_chunks, _IDX_CHUNK), jnp.int32),
          pltpu.VMEM((b_per_w, W), jnp.float32),
          pltpu.SemaphoreType.DMA,
      ],
  )
  def gather_k(packed_hbm, idx_hbm, out_hbm, idx_v, panel, sem):
    wid = lax.axis_index("s") * _NC + lax.axis_index("c")
    base = wid * b_per_w
    pltpu.sync_copy(idx_hbm.at[wid], idx_v)
    copies = []
    for j in range(n_chunks):
      copies.append(
          pltpu.async_copy(
              packed_hbm.at[idx_v.at[j]],
              panel.at[pl.ds(j * _IDX_CHUNK, _IDX_CHUNK)],
              sem,
          )
      )
    for c in copies:
      c.wait()
    pltpu.sync_copy(panel, out_hbm.at[pl.ds(base, b_per_w)])

  return gather_k


def _gru_body(emb2_ref, par_ref, h_ref, wi_ref, wh_ref, bi_ref, bh_ref,
              out_ref):
  emb2 = emb2_ref[...]
  par = par_ref[...]
  emb = jnp.where(par == 0, emb2[:, :64], emb2[:, 64:])
  h = h_ref[...]
  f32 = jnp.float32
  gi_r = jnp.dot(emb, wi_ref[0], preferred_element_type=f32) + bi_ref[0]
  gi_z = jnp.dot(emb, wi_ref[1], preferred_element_type=f32) + bi_ref[1]
  gi_n = jnp.dot(emb, wi_ref[2], preferred_element_type=f32) + bi_ref[2]
  gh_r = jnp.dot(h, wh_ref[0], preferred_element_type=f32) + bh_ref[0]
  gh_z = jnp.dot(h, wh_ref[1], preferred_element_type=f32) + bh_ref[1]
  gh_n = jnp.dot(h, wh_ref[2], preferred_element_type=f32) + bh_ref[2]
  r = jax.nn.sigmoid(gi_r + gh_r)
  z = jax.nn.sigmoid(gi_z + gh_z)
  n = jnp.tanh(gi_n + r * gh_n)
  out_ref[...] = (1.0 - z) * n + z * h


def _gru_call(emb2, par, h, wi, wh, bi, bh, blk):
  B, H = h.shape
  grid = B // blk
  return pl.pallas_call(
      _gru_body,
      grid=(grid,),
      in_specs=[
          pl.BlockSpec((blk, emb2.shape[1]), lambda i: (i, 0)),
          pl.BlockSpec((blk, 1), lambda i: (i, 0)),
          pl.BlockSpec((blk, H), lambda i: (i, 0)),
          pl.BlockSpec(wi.shape, lambda i: (0, 0, 0)),
          pl.BlockSpec(wh.shape, lambda i: (0, 0, 0)),
          pl.BlockSpec(bi.shape, lambda i: (0, 0, 0)),
          pl.BlockSpec(bh.shape, lambda i: (0, 0, 0)),
      ],
      out_specs=pl.BlockSpec((blk, H), lambda i: (i, 0)),
      out_shape=jax.ShapeDtypeStruct((B, H), jnp.float32),
  )(emb2, par, h, wi, wh, bi, bh)


@jax.jit
def kernel(x, h, table, W_ih, W_hh, b_ih, b_hh):
  V, D = table.shape
  B, H = h.shape

  packed = _repack(table.T, V, D)

  xi = x.astype(jnp.int32)
  half = _TL // 2
  b = xi // _TL
  q = xi % _TL
  idx2 = (b * half + q % half).reshape(
      _NW, (B // _NW) // _IDX_CHUNK, _IDX_CHUNK
  )
  par = (q // half).reshape(B, 1)

  emb2 = _make_sc_gather(packed.shape[0], 2 * D, B)(packed, idx2)

  wi = jnp.transpose(W_ih.reshape(3, H, D), (0, 2, 1))
  wh = jnp.transpose(W_hh.reshape(3, H, H), (0, 2, 1))
  bi = b_ih.reshape(3, 1, H)
  bh = b_hh.reshape(3, 1, H)

  return _gru_call(emb2, par, h, wi, wh, bi, bh, blk=4096)
